# baseline (speedup 1.0000x reference)
import jax
import jax.numpy as jnp
from jax import lax
from jax.experimental import pallas as pl
from jax.experimental.pallas import tpu as pltpu
from jax.experimental.pallas import tpu_sc as plsc

B = 16384
D = 128
K = 20
KP1 = K + 1
NW = 32
BPW = B // NW
CH = 16
NCHUNK = BPW // CH
RPC = CH * KP1
SPW = BPW * KP1


def _sc_scores_kernel(cenidx_hbm, comb_hbm, cen_tab_hbm, ctx_tab_hbm,
                      sc_hbm,
                      cen_idx_v, nidx_a, nidx_b,
                      cen_a, cen_b, rows_a, rows_b,
                      sc_buf, sem_a, sem_b, sem_ia, sem_ib):
  wid = lax.axis_index("s") * 2 + lax.axis_index("c")
  base = wid * BPW

  pltpu.sync_copy(cenidx_hbm.at[pl.ds(base, BPW)], cen_idx_v)

  def stage_desc(g, idx_buf, sem):
    return pltpu.make_async_copy(
        comb_hbm.at[pl.ds(base + g * CH, CH), :], idx_buf, sem)

  def gather_descs(g, cen_buf, rows_buf, idx_buf, sem):
    descs = [
        pltpu.make_async_copy(
            cen_tab_hbm.at[cen_idx_v.at[pl.ds(g * CH, CH)]], cen_buf, sem),
    ]
    for b in range(CH):
      descs.append(pltpu.make_async_copy(
          ctx_tab_hbm.at[idx_buf.at[b, pl.ds(0, KP1)]],
          rows_buf.at[pl.ds(b * KP1, KP1)], sem))
    return descs

  def issue(g, bufs, sem):
    for d in gather_descs(g, *bufs, sem):
      d.start()

  def drain(g, bufs, sem):
    for d in gather_descs(g, *bufs, sem):
      d.wait()

  lane = lax.broadcasted_iota(jnp.int32, (16,), 0)
  shifts = [(lane + s) & 15 for s in (8, 4, 2, 1)]

  def compute(g, bufs):
    cen_buf, rows_buf, _ = bufs

    def b_body(b, c):
      c_nxt = tuple(cen_buf[jnp.minimum(b + 1, CH - 1), pl.ds(16 * j, 16)]
                    for j in range(8))
      sbase = (g * CH + b) * KP1
      coll_a = coll_b = None
      for j2 in range(KP1):
        r = b * KP1 + j2
        p = [c[j] * rows_buf[r, pl.ds(16 * j, 16)] for j in range(8)]
        q = [p[0] + p[1], p[2] + p[3], p[4] + p[5], p[6] + p[7]]
        acc = (q[0] + q[1]) + (q[2] + q[3])
        for sh in shifts:
          acc = acc + acc.at[sh].get(mode="promise_in_bounds")
        if j2 == 0:
          coll_a = acc
        elif j2 < 16:
          coll_a = jnp.where(lane == j2, acc, coll_a)
        elif j2 == 16:
          coll_b = acc
        else:
          coll_b = jnp.where(lane == (j2 - 16), acc, coll_b)
      sc_buf[pl.ds(sbase, 16)] = coll_a
      sc_buf[pl.ds(sbase + 16, 16)] = coll_b
      return c_nxt
    c0 = tuple(cen_buf[0, pl.ds(16 * j, 16)] for j in range(8))
    lax.fori_loop(0, CH, b_body, c0)

  bufs_a = (cen_a, rows_a, nidx_a)
  bufs_b = (cen_b, rows_b, nidx_b)

  s0 = stage_desc(0, nidx_a, sem_ia)
  s0.start()
  s0.wait()
  issue(0, bufs_a, sem_a)
  stage_desc(1, nidx_b, sem_ib).start()

  def body2(i, carry):
    g = 2 * i
    stage_desc(g + 1, nidx_b, sem_ib).wait()
    issue(g + 1, bufs_b, sem_b)
    drain(g, bufs_a, sem_a)

    @pl.when(g + 2 < NCHUNK)
    def _():
      stage_desc(g + 2, nidx_a, sem_ia).start()

    compute(g, bufs_a)

    @pl.when(g + 2 < NCHUNK)
    def _():
      stage_desc(g + 2, nidx_a, sem_ia).wait()
      issue(g + 2, bufs_a, sem_a)

    @pl.when(g + 3 < NCHUNK)
    def _():
      stage_desc(g + 3, nidx_b, sem_ib).start()

    drain(g + 1, bufs_b, sem_b)
    compute(g + 1, bufs_b)
    return carry

  lax.fori_loop(0, NCHUNK // 2, body2, 0)

  pltpu.sync_copy(sc_buf.at[pl.ds(0, SPW)], sc_hbm.at[pl.ds(base * KP1, SPW)])


def _sc_scores(cen_idx, comb_pad, cen_tab, ctx_tab):
  mesh = plsc.VectorSubcoreMesh(core_axis_name="c", subcore_axis_name="s")
  f = pl.kernel(
      _sc_scores_kernel,
      out_type=jax.ShapeDtypeStruct((B * KP1,), jnp.float32),
      mesh=mesh,
      scratch_types=[
          pltpu.VMEM((BPW,), jnp.int32),
          pltpu.VMEM((CH, D), jnp.int32),
          pltpu.VMEM((CH, D), jnp.int32),
          pltpu.VMEM((CH, D), jnp.float32),
          pltpu.VMEM((CH, D), jnp.float32),
          pltpu.VMEM((RPC, D), jnp.float32),
          pltpu.VMEM((RPC, D), jnp.float32),
          pltpu.VMEM((SPW + 16,), jnp.float32),
          pltpu.SemaphoreType.DMA,
          pltpu.SemaphoreType.DMA,
          pltpu.SemaphoreType.DMA,
          pltpu.SemaphoreType.DMA,
      ],
      compiler_params=pltpu.CompilerParams(needs_layout_passes=False),
  )
  return f(cen_idx, comb_pad, cen_tab, ctx_tab)


def _loss_body(sc_ref, out_ref):
  x = jnp.reshape(sc_ref[...], (B * KP1 // D, D))
  gidx = (lax.broadcasted_iota(jnp.int32, x.shape, 0) * D
          + lax.broadcasted_iota(jnp.int32, x.shape, 1))
  is_pos = (gidx % KP1) == 0
  w = jnp.where(is_pos, 1.0 / B, 1.0 / (B * K))
  sp = jnp.logaddexp(x, 0.0)
  total = (jnp.sum(sp * w)
           - jnp.sum(jnp.where(is_pos, x, 0.0)) * (1.0 / B))
  out_ref[...] = jnp.reshape(total, (1, 1))


def _loss(sc_flat):
  return pl.pallas_call(
      _loss_body,
      out_shape=jax.ShapeDtypeStruct((1, 1), jnp.float32),
  )(sc_flat)


def kernel(center_word_indices, context_word_indices, negative_word_indices,
           center_table, context_table):
  cen_idx = center_word_indices.astype(jnp.int32)
  ctx_idx = context_word_indices.astype(jnp.int32)
  neg_idx = negative_word_indices.astype(jnp.int32)
  comb_pad = jnp.concatenate(
      [ctx_idx[:, None], neg_idx,
       jnp.zeros((B, D - KP1), jnp.int32)], axis=1)
  scores = _sc_scores(cen_idx, comb_pad, center_table, context_table)
  loss = _loss(scores)
  return loss[0, 0]


# speedup vs baseline: 1.0713x; 1.0713x over previous
import jax
import jax.numpy as jnp
from jax import lax
from jax.experimental import pallas as pl
from jax.experimental.pallas import tpu as pltpu
from jax.experimental.pallas import tpu_sc as plsc

B = 16384
D = 128
K = 20
KP1 = K + 1
NW = 32
BPW = B // NW
CH = 16
NCHUNK = BPW // CH
RPC = CH * KP1
SPW = BPW * KP1


def _sc_scores_kernel(cenidx_hbm, comb_hbm, cen_tab_hbm, ctx_tab_hbm,
                      sc_hbm,
                      cen_idx_v, nidx_a, nidx_b,
                      cen_a, cen_b, rows_a, rows_b,
                      sc_buf, sem_a, sem_b, sem_ia, sem_ib):
  wid = lax.axis_index("s") * 2 + lax.axis_index("c")
  base = wid * BPW

  pltpu.sync_copy(cenidx_hbm.at[pl.ds(base, BPW)], cen_idx_v)

  def stage_desc(g, idx_buf, sem):
    return pltpu.make_async_copy(
        comb_hbm.at[pl.ds(base + g * CH, CH), :], idx_buf, sem)

  def gather_descs(g, cen_buf, rows_buf, idx_buf, sem):
    descs = [
        pltpu.make_async_copy(
            cen_tab_hbm.at[cen_idx_v.at[pl.ds(g * CH, CH)]], cen_buf, sem),
    ]
    for b in range(CH):
      descs.append(pltpu.make_async_copy(
          ctx_tab_hbm.at[idx_buf.at[b, pl.ds(0, KP1)]],
          rows_buf.at[pl.ds(b * KP1, KP1)], sem))
    return descs

  def issue(g, bufs, sem):
    for d in gather_descs(g, *bufs, sem):
      d.start()

  def drain(g, bufs, sem):
    for d in gather_descs(g, *bufs, sem):
      d.wait()

  lane = lax.broadcasted_iota(jnp.int32, (16,), 0)
  shifts = [(lane + s) & 15 for s in (8, 4, 2, 1)]

  def compute(g, bufs):
    cen_buf, rows_buf, _ = bufs

    def b_body(b, c):
      c_nxt = tuple(cen_buf[jnp.minimum(b + 1, CH - 1), pl.ds(16 * j, 16)]
                    for j in range(8))
      sbase = (g * CH + b) * KP1
      coll_a = coll_b = None
      for j2 in range(KP1):
        r = b * KP1 + j2
        p = [c[j] * rows_buf[r, pl.ds(16 * j, 16)] for j in range(8)]
        q = [p[0] + p[1], p[2] + p[3], p[4] + p[5], p[6] + p[7]]
        acc = (q[0] + q[1]) + (q[2] + q[3])
        for sh in shifts:
          acc = acc + acc.at[sh].get(mode="promise_in_bounds")
        if j2 == 0:
          coll_a = acc
        elif j2 < 16:
          coll_a = jnp.where(lane == j2, acc, coll_a)
        elif j2 == 16:
          coll_b = acc
        else:
          coll_b = jnp.where(lane == (j2 - 16), acc, coll_b)
      sc_buf[pl.ds(sbase, 16)] = coll_a
      sc_buf[pl.ds(sbase + 16, 16)] = coll_b
      return c_nxt
    c0 = tuple(cen_buf[0, pl.ds(16 * j, 16)] for j in range(8))
    lax.fori_loop(0, CH, b_body, c0)

  bufs_a = (cen_a, rows_a, nidx_a)
  bufs_b = (cen_b, rows_b, nidx_b)

  s0 = stage_desc(0, nidx_a, sem_ia)
  s0.start()
  s0.wait()
  issue(0, bufs_a, sem_a)
  stage_desc(1, nidx_b, sem_ib).start()

  def body2(i, carry):
    g = 2 * i
    stage_desc(g + 1, nidx_b, sem_ib).wait()
    issue(g + 1, bufs_b, sem_b)
    drain(g, bufs_a, sem_a)

    @pl.when(g + 2 < NCHUNK)
    def _():
      stage_desc(g + 2, nidx_a, sem_ia).start()

    compute(g, bufs_a)

    @pl.when(g + 2 < NCHUNK)
    def _():
      stage_desc(g + 2, nidx_a, sem_ia).wait()
      issue(g + 2, bufs_a, sem_a)

    @pl.when(g + 3 < NCHUNK)
    def _():
      stage_desc(g + 3, nidx_b, sem_ib).start()

    drain(g + 1, bufs_b, sem_b)
    compute(g + 1, bufs_b)
    return carry

  lax.fori_loop(0, NCHUNK // 2, body2, 0)

  pltpu.sync_copy(sc_buf.at[pl.ds(0, SPW)], sc_hbm.at[pl.ds(base * KP1, SPW)])


def _sc_scores(cen_idx, comb_pad, cen_tab, ctx_tab):
  mesh = plsc.VectorSubcoreMesh(core_axis_name="c", subcore_axis_name="s")
  f = pl.kernel(
      _sc_scores_kernel,
      out_type=jax.ShapeDtypeStruct((B * KP1,), jnp.float32),
      mesh=mesh,
      scratch_types=[
          pltpu.VMEM((BPW,), jnp.int32),
          pltpu.VMEM((CH, D), jnp.int32),
          pltpu.VMEM((CH, D), jnp.int32),
          pltpu.VMEM((CH, D), jnp.float32),
          pltpu.VMEM((CH, D), jnp.float32),
          pltpu.VMEM((RPC, D), jnp.float32),
          pltpu.VMEM((RPC, D), jnp.float32),
          pltpu.VMEM((SPW + 16,), jnp.float32),
          pltpu.SemaphoreType.DMA,
          pltpu.SemaphoreType.DMA,
          pltpu.SemaphoreType.DMA,
          pltpu.SemaphoreType.DMA,
      ],
      compiler_params=pltpu.CompilerParams(needs_layout_passes=False),
  )
  return f(cen_idx, comb_pad, cen_tab, ctx_tab)


def _loss_body(sc_ref, out_ref):
  x = jnp.reshape(sc_ref[...], (B * KP1 // D, D))
  gidx = (lax.broadcasted_iota(jnp.int32, x.shape, 0) * D
          + lax.broadcasted_iota(jnp.int32, x.shape, 1))
  is_pos = (gidx % KP1) == 0
  w = jnp.where(is_pos, 1.0 / B, 1.0 / (B * K))
  sp = jnp.logaddexp(x, 0.0)
  total = (jnp.sum(sp * w)
           - jnp.sum(jnp.where(is_pos, x, 0.0)) * (1.0 / B))
  out_ref[...] = jnp.reshape(total, (1, 1))


def _loss(sc_flat):
  return pl.pallas_call(
      _loss_body,
      out_shape=jax.ShapeDtypeStruct((1, 1), jnp.float32),
  )(sc_flat)


def kernel(center_word_indices, context_word_indices, negative_word_indices,
           center_table, context_table):
  cen_idx = center_word_indices.astype(jnp.int32)
  ctx_idx = context_word_indices.astype(jnp.int32)
  neg_idx = negative_word_indices.astype(jnp.int32)
  comb = jnp.concatenate([ctx_idx[:, None], neg_idx], axis=1)
  comb_pad = jnp.pad(comb, ((0, 0), (0, D - KP1)))
  scores = _sc_scores(cen_idx, comb_pad, center_table, context_table)
  loss = _loss(scores)
  return loss[0, 0]
